# Initial kernel scaffold; baseline (speedup 1.0000x reference)
import jax
import jax.numpy as jnp
from jax.experimental import pallas as pl


def kernel(X, edge_index, batch, params):
    raise NotImplementedError("write your pallas kernel here")


# speedup vs baseline: 42.0289x; 42.0289x over previous
import functools

import numpy as np
import jax
import jax.numpy as jnp
from jax import lax
from jax.experimental import pallas as pl
from jax.experimental.pallas import tpu as pltpu
from jax.experimental.pallas import tpu_sc as plsc

N = 50000
E = 800000
B = 64

_NC = 2
_NS = 16
_NW = _NC * _NS

_MSUM = np.zeros((64, 8), np.float32)
for _i in range(64):
    _MSUM[_i, _i // 8] = 1.0
_MEXP = np.ascontiguousarray(_MSUM.T)

BN_ROWS = 2000
NB = N // BN_ROWS
BE = 2000
GCH = 200
EPW = E // _NW
GIT = EPW // GCH
EPT = E // _NS
AIT = EPT // GCH
ZCH = 400
ZNB = N // ZCH
ZPT = -(-ZNB // _NS)

_sc_mesh = plsc.VectorSubcoreMesh(core_axis_name="c", subcore_axis_name="s")
_sc_params = pltpu.CompilerParams(use_tc_tiling_on_sc=False)


def _qkvs_body(x_ref, w_ref, b_ref, q_ref, kv_ref, s_ref):
    out = jnp.dot(x_ref[...], w_ref[...],
                  preferred_element_type=jnp.float32) + b_ref[...]
    q_ref[...] = out[:, :64]
    kv_ref[...] = out[:, 64:192]
    s_ref[...] = out[:, 192:256]


def _qkvs(x, w, b):
    din = x.shape[1]
    return pl.pallas_call(
        _qkvs_body,
        grid=(NB,),
        in_specs=[pl.BlockSpec((BN_ROWS, din), lambda i: (i, 0)),
                  pl.BlockSpec((din, 256), lambda i: (0, 0)),
                  pl.BlockSpec((1, 256), lambda i: (0, 0))],
        out_specs=[pl.BlockSpec((BN_ROWS, 64), lambda i: (i, 0)),
                   pl.BlockSpec((BN_ROWS, 128), lambda i: (i, 0)),
                   pl.BlockSpec((BN_ROWS, 64), lambda i: (i, 0))],
        out_shape=[jax.ShapeDtypeStruct((N, 64), jnp.float32),
                   jax.ShapeDtypeStruct((N, 128), jnp.float32),
                   jax.ShapeDtypeStruct((N, 64), jnp.float32)],
    )(x, w, b)


def _gather_body(q_hbm, kv_hbm, dst_hbm, src_hbm, qd_hbm, kvs_hbm,
                 dst_v, src_v, qrows, kvrows, sem1, sem2):
    cid = lax.axis_index("c")
    sid = lax.axis_index("s")
    wid = sid * _NC + cid
    base = wid * EPW

    def body(i, carry):
        off = base + i * GCH
        pltpu.sync_copy(dst_hbm.at[pl.ds(off, GCH)], dst_v)
        pltpu.sync_copy(src_hbm.at[pl.ds(off, GCH)], src_v)
        cp1 = pltpu.async_copy(q_hbm.at[dst_v], qrows, sem1)
        cp2 = pltpu.async_copy(kv_hbm.at[src_v], kvrows, sem2)
        cp1.wait()
        cp2.wait()
        pltpu.sync_copy(qrows, qd_hbm.at[pl.ds(off, GCH)])
        pltpu.sync_copy(kvrows, kvs_hbm.at[pl.ds(off, GCH)])
        return carry

    lax.fori_loop(0, GIT, body, 0)


_gather = pl.kernel(
    _gather_body,
    out_type=[jax.ShapeDtypeStruct((E, 64), jnp.float32),
              jax.ShapeDtypeStruct((E, 128), jnp.float32)],
    mesh=_sc_mesh,
    scratch_types=[pltpu.VMEM((GCH,), jnp.int32),
                   pltpu.VMEM((GCH,), jnp.int32),
                   pltpu.VMEM((GCH, 64), jnp.float32),
                   pltpu.VMEM((GCH, 128), jnp.float32),
                   pltpu.SemaphoreType.DMA,
                   pltpu.SemaphoreType.DMA],
    compiler_params=_sc_params,
)


def _edge_body(qd_ref, kvs_ref, ms_ref, me_ref, ex_ref, ua_ref, ub_ref):
    qd = qd_ref[...]
    k = kvs_ref[:, :64]
    v = kvs_ref[:, 64:]
    alpha = jnp.dot(qd * k, ms_ref[...],
                    preferred_element_type=jnp.float32, precision=lax.Precision.HIGHEST) * (1.0 / np.sqrt(8.0))
    ex = jnp.exp(alpha)
    ex_ref[...] = jnp.concatenate([ex, jnp.zeros_like(ex)], axis=1)
    u = v * jnp.dot(ex, me_ref[...], preferred_element_type=jnp.float32, precision=lax.Precision.HIGHEST)
    ua_ref[...] = u[:, :32]
    ub_ref[...] = u[:, 32:]


def _edge(qd, kvs, msum, mexp):
    return pl.pallas_call(
        _edge_body,
        grid=(E // BE,),
        in_specs=[pl.BlockSpec((BE, 64), lambda i: (i, 0)),
                  pl.BlockSpec((BE, 128), lambda i: (i, 0)),
                  pl.BlockSpec((64, 8), lambda i: (0, 0)),
                  pl.BlockSpec((8, 64), lambda i: (0, 0))],
        out_specs=[pl.BlockSpec((BE, 16), lambda i: (i, 0)),
                   pl.BlockSpec((BE, 32), lambda i: (i, 0)),
                   pl.BlockSpec((BE, 32), lambda i: (i, 0))],
        out_shape=[jax.ShapeDtypeStruct((E, 16), jnp.float32),
                   jax.ShapeDtypeStruct((E, 32), jnp.float32),
                   jax.ShapeDtypeStruct((E, 32), jnp.float32)],
    )(qd, kvs, msum, mexp)


def _denom_body(ex_hbm, dst_hbm, z_hbm, d0_hbm, d1_hbm,
                dst_v, rows, wbuf, acc, sem):
    cid = lax.axis_index("c")
    sid = lax.axis_index("s")
    wid = sid * _NC + cid
    pltpu.sync_copy(z_hbm, wbuf)

    def zbody(t, carry):
        j = sid + t * _NS

        @pl.when(j < ZNB)
        def _():
            pltpu.sync_copy(wbuf, acc.at[pl.ds(j * ZCH, ZCH)])
        return carry

    lax.fori_loop(0, ZPT, zbody, 0)
    plsc.subcore_barrier()

    base = wid * EPW

    def body(i, carry):
        off = base + i * GCH
        pltpu.sync_copy(dst_hbm.at[pl.ds(off, GCH)], dst_v)
        pltpu.sync_copy(ex_hbm.at[pl.ds(off, GCH)], rows)
        pltpu.sync_copy(rows, acc.at[dst_v], add=True)
        return carry

    lax.fori_loop(0, GIT, body, 0)
    plsc.subcore_barrier()

    def wbody(t, carry):
        j = sid + t * _NS

        @pl.when(j < ZNB)
        def _():
            r0 = j * ZCH
            pltpu.sync_copy(acc.at[pl.ds(r0, ZCH)], wbuf)

            @pl.when(cid == 0)
            def _():
                pltpu.sync_copy(wbuf, d0_hbm.at[pl.ds(r0, ZCH)])

            @pl.when(cid == 1)
            def _():
                pltpu.sync_copy(wbuf, d1_hbm.at[pl.ds(r0, ZCH)])
        return carry

    lax.fori_loop(0, ZPT, wbody, 0)


_denom = pl.kernel(
    _denom_body,
    out_type=[jax.ShapeDtypeStruct((N, 16), jnp.float32),
              jax.ShapeDtypeStruct((N, 16), jnp.float32)],
    mesh=_sc_mesh,
    scratch_types=[pltpu.VMEM((GCH,), jnp.int32),
                   pltpu.VMEM((GCH, 16), jnp.float32),
                   pltpu.VMEM((ZCH, 16), jnp.float32),
                   pltpu.VMEM_SHARED((N, 16), jnp.float32),
                   pltpu.SemaphoreType.DMA],
    compiler_params=_sc_params,
)


def _agg_body(ua_hbm, ub_hbm, dst_hbm, z_hbm, a0_hbm, a1_hbm,
              dst_v, rows, wbuf, acc, sem):
    cid = lax.axis_index("c")
    sid = lax.axis_index("s")
    pltpu.sync_copy(z_hbm, wbuf)

    def zbody(t, carry):
        j = sid + t * _NS

        @pl.when(j < ZNB)
        def _():
            pltpu.sync_copy(wbuf, acc.at[pl.ds(j * ZCH, ZCH)])
        return carry

    lax.fori_loop(0, ZPT, zbody, 0)
    plsc.subcore_barrier()

    base = sid * EPT

    def body(i, carry):
        off = base + i * GCH
        pltpu.sync_copy(dst_hbm.at[pl.ds(off, GCH)], dst_v)

        @pl.when(cid == 0)
        def _():
            pltpu.sync_copy(ua_hbm.at[pl.ds(off, GCH)], rows)

        @pl.when(cid == 1)
        def _():
            pltpu.sync_copy(ub_hbm.at[pl.ds(off, GCH)], rows)

        pltpu.sync_copy(rows, acc.at[dst_v], add=True)
        return carry

    lax.fori_loop(0, AIT, body, 0)
    plsc.subcore_barrier()

    def wbody(t, carry):
        j = sid + t * _NS

        @pl.when(j < ZNB)
        def _():
            r0 = j * ZCH
            pltpu.sync_copy(acc.at[pl.ds(r0, ZCH)], wbuf)

            @pl.when(cid == 0)
            def _():
                pltpu.sync_copy(wbuf, a0_hbm.at[pl.ds(r0, ZCH)])

            @pl.when(cid == 1)
            def _():
                pltpu.sync_copy(wbuf, a1_hbm.at[pl.ds(r0, ZCH)])
        return carry

    lax.fori_loop(0, ZPT, wbody, 0)


_agg = pl.kernel(
    _agg_body,
    out_type=[jax.ShapeDtypeStruct((N, 32), jnp.float32),
              jax.ShapeDtypeStruct((N, 32), jnp.float32)],
    mesh=_sc_mesh,
    scratch_types=[pltpu.VMEM((GCH,), jnp.int32),
                   pltpu.VMEM((GCH, 32), jnp.float32),
                   pltpu.VMEM((ZCH, 32), jnp.float32),
                   pltpu.VMEM_SHARED((N, 32), jnp.float32),
                   pltpu.SemaphoreType.DMA],
    compiler_params=_sc_params,
)


def _post_a_body(a0_ref, a1_ref, d0_ref, d1_ref, s_ref, me_ref,
                 r_ref, ps_ref, pq_ref):
    d8 = jnp.maximum(d0_ref[:, :8] + d1_ref[:, :8], 1e-30)
    dexp = jnp.dot(d8, me_ref[...], preferred_element_type=jnp.float32, precision=lax.Precision.HIGHEST)
    agg = jnp.concatenate([a0_ref[...], a1_ref[...]], axis=1) / dexp
    r = jnp.maximum(agg + s_ref[...], 0.0)
    r_ref[...] = r
    ps_ref[...] = jnp.sum(r, axis=0, keepdims=True).reshape(1, 1, 64)
    pq_ref[...] = jnp.sum(r * r, axis=0, keepdims=True).reshape(1, 1, 64)


def _post_a(a0, a1, d0, d1, s, mexp):
    return pl.pallas_call(
        _post_a_body,
        grid=(NB,),
        in_specs=[pl.BlockSpec((BN_ROWS, 32), lambda i: (i, 0)),
                  pl.BlockSpec((BN_ROWS, 32), lambda i: (i, 0)),
                  pl.BlockSpec((BN_ROWS, 16), lambda i: (i, 0)),
                  pl.BlockSpec((BN_ROWS, 16), lambda i: (i, 0)),
                  pl.BlockSpec((BN_ROWS, 64), lambda i: (i, 0)),
                  pl.BlockSpec((8, 64), lambda i: (0, 0))],
        out_specs=[pl.BlockSpec((BN_ROWS, 64), lambda i: (i, 0)),
                   pl.BlockSpec((1, 1, 64), lambda i: (i, 0, 0)),
                   pl.BlockSpec((1, 1, 64), lambda i: (i, 0, 0))],
        out_shape=[jax.ShapeDtypeStruct((N, 64), jnp.float32),
                   jax.ShapeDtypeStruct((NB, 1, 64), jnp.float32),
                   jax.ShapeDtypeStruct((NB, 1, 64), jnp.float32)],
    )(a0, a1, d0, d1, s, mexp)


def _post_b_body_res(r_ref, ps_ref, pq_ref, g_ref, b_ref, xp_ref, o_ref):
    mu = jnp.sum(ps_ref[...].reshape(NB, 64), axis=0, keepdims=True) * (1.0 / N)
    msq = jnp.sum(pq_ref[...].reshape(NB, 64), axis=0, keepdims=True) * (1.0 / N)
    var = msq - mu * mu
    inv = lax.rsqrt(var + 1e-5)
    o_ref[...] = ((r_ref[...] - mu) * inv * g_ref[...] + b_ref[...]
                  + xp_ref[...])


def _post_b_body_nores(r_ref, ps_ref, pq_ref, g_ref, b_ref, o_ref):
    mu = jnp.sum(ps_ref[...].reshape(NB, 64), axis=0, keepdims=True) * (1.0 / N)
    msq = jnp.sum(pq_ref[...].reshape(NB, 64), axis=0, keepdims=True) * (1.0 / N)
    var = msq - mu * mu
    inv = lax.rsqrt(var + 1e-5)
    o_ref[...] = (r_ref[...] - mu) * inv * g_ref[...] + b_ref[...]


def _post_b(r, ps, pq, gamma, beta, xprev):
    body = _post_b_body_nores if xprev is None else _post_b_body_res
    in_specs = [pl.BlockSpec((BN_ROWS, 64), lambda i: (i, 0)),
                pl.BlockSpec((NB, 1, 64), lambda i: (0, 0, 0)),
                pl.BlockSpec((NB, 1, 64), lambda i: (0, 0, 0)),
                pl.BlockSpec((1, 64), lambda i: (0, 0)),
                pl.BlockSpec((1, 64), lambda i: (0, 0))]
    args = [r, ps, pq, gamma, beta]
    if xprev is not None:
        in_specs.append(pl.BlockSpec((BN_ROWS, 64), lambda i: (i, 0)))
        args.append(xprev)
    return pl.pallas_call(
        body,
        grid=(NB,),
        in_specs=in_specs,
        out_specs=pl.BlockSpec((BN_ROWS, 64), lambda i: (i, 0)),
        out_shape=jax.ShapeDtypeStruct((N, 64), jnp.float32),
    )(*args)


def _pool_body(x_ref, b_ref, s_ref, c_ref):
    i = pl.program_id(0)

    @pl.when(i == 0)
    def _():
        s_ref[...] = jnp.zeros_like(s_ref)
        c_ref[...] = jnp.zeros_like(c_ref)

    bvals = b_ref[...]
    cols = lax.broadcasted_iota(jnp.int32, (BN_ROWS, B), 1)
    onehot = jnp.where(bvals == cols, 1.0, 0.0)
    s_ref[...] += lax.dot_general(onehot, x_ref[...],
                                  (((0,), (0,)), ((), ())),
                                  preferred_element_type=jnp.float32,
                                  precision=lax.Precision.HIGHEST)
    c_ref[...] += lax.dot_general(onehot, jnp.ones((BN_ROWS, 1), jnp.float32),
                                  (((0,), (0,)), ((), ())),
                                  preferred_element_type=jnp.float32,
                                  precision=lax.Precision.HIGHEST)


def _pool(x, batch2d):
    return pl.pallas_call(
        _pool_body,
        grid=(NB,),
        in_specs=[pl.BlockSpec((BN_ROWS, 64), lambda i: (i, 0)),
                  pl.BlockSpec((BN_ROWS, 1), lambda i: (i, 0))],
        out_specs=[pl.BlockSpec((B, 64), lambda i: (0, 0)),
                   pl.BlockSpec((B, 1), lambda i: (0, 0))],
        out_shape=[jax.ShapeDtypeStruct((B, 64), jnp.float32),
                   jax.ShapeDtypeStruct((B, 1), jnp.float32)],
    )(x, batch2d)


def _final_body(s_ref, c_ref, w_ref, b_ref, o_ref, xm_ref):
    xm = s_ref[...] / jnp.maximum(c_ref[...], 1.0)
    xm_ref[...] = xm
    o_ref[...] = jnp.dot(xm, w_ref[...],
                         preferred_element_type=jnp.float32) + b_ref[...]


def _final(ssum, cnt, w, b):
    return pl.pallas_call(
        _final_body,
        grid=(1,),
        in_specs=[pl.BlockSpec((B, 64), lambda i: (0, 0)),
                  pl.BlockSpec((B, 1), lambda i: (0, 0)),
                  pl.BlockSpec((64, 1), lambda i: (0, 0)),
                  pl.BlockSpec((1, 1), lambda i: (0, 0))],
        out_specs=[pl.BlockSpec((B, 1), lambda i: (0, 0)),
                   pl.BlockSpec((B, 64), lambda i: (0, 0))],
        out_shape=[jax.ShapeDtypeStruct((B, 1), jnp.float32),
                   jax.ShapeDtypeStruct((B, 64), jnp.float32)],
    )(ssum, cnt, w, b)


def kernel(X, edge_index, batch, params):
    src = edge_index[0]
    dst = edge_index[1]
    msum = jnp.asarray(_MSUM)
    mexp = jnp.asarray(_MEXP)
    z16 = jnp.zeros((ZCH, 16), jnp.float32)
    z32 = jnp.zeros((ZCH, 32), jnp.float32)

    x = X
    for l in range(5):
        p = params['convs'][l]
        w_all = jnp.concatenate([p['Wq'], p['Wk'], p['Wv'], p['Ws']], axis=1)
        b_all = jnp.concatenate([p['bq'], p['bk'], p['bv'], p['bs']]).reshape(1, 256)
        q, kv, s = _qkvs(x, w_all, b_all)
        qd, kvs = _gather(q, kv, dst, src)
        ex16, ua, ub = _edge(qd, kvs, msum, mexp)
        d0, d1 = _denom(ex16, dst, z16)
        a0, a1 = _agg(ua, ub, dst, z32)
        r, ps, pq = _post_a(a0, a1, d0, d1, s, mexp)
        bn = params['bns'][l]
        x = _post_b(r, ps, pq, bn['gamma'].reshape(1, 64), bn['beta'].reshape(1, 64),
                    xprev=(x if l > 0 else None))

    ssum, cnt = _pool(x, batch.reshape(N, 1))
    out, xm = _final(ssum, cnt, params['reg_W'], params['reg_b'].reshape(1, 1))
    return (out, xm)


# speedup vs baseline: 48.4308x; 1.1523x over previous
import functools

import numpy as np
import jax
import jax.numpy as jnp
from jax import lax
from jax.experimental import pallas as pl
from jax.experimental.pallas import tpu as pltpu
from jax.experimental.pallas import tpu_sc as plsc

N = 50000
E = 800000
B = 64

_NC = 2
_NS = 16
_NW = _NC * _NS

_MSUM = np.zeros((64, 8), np.float32)
for _i in range(64):
    _MSUM[_i, _i // 8] = 1.0
_MEXP = np.ascontiguousarray(_MSUM.T)

BN_ROWS = 2000
NB = N // BN_ROWS
BE = 2000
GCH = 200
EPW = E // _NW
GIT = EPW // GCH
EPT = E // _NS
AIT = EPT // GCH
ZCH = 400
ZNB = N // ZCH
ZPT = -(-ZNB // _NS)

_sc_mesh = plsc.VectorSubcoreMesh(core_axis_name="c", subcore_axis_name="s")
_sc_params = pltpu.CompilerParams(use_tc_tiling_on_sc=False)


def _qkvs_body(x_ref, w_ref, b_ref, q_ref, kv_ref, s_ref):
    out = jnp.dot(x_ref[...], w_ref[...],
                  preferred_element_type=jnp.float32) + b_ref[...]
    q_ref[...] = out[:, :64]
    kv_ref[...] = out[:, 64:192]
    s_ref[...] = out[:, 192:256]


def _qkvs(x, w, b):
    din = x.shape[1]
    return pl.pallas_call(
        _qkvs_body,
        grid=(NB,),
        in_specs=[pl.BlockSpec((BN_ROWS, din), lambda i: (i, 0)),
                  pl.BlockSpec((din, 256), lambda i: (0, 0)),
                  pl.BlockSpec((1, 256), lambda i: (0, 0))],
        out_specs=[pl.BlockSpec((BN_ROWS, 64), lambda i: (i, 0)),
                   pl.BlockSpec((BN_ROWS, 128), lambda i: (i, 0)),
                   pl.BlockSpec((BN_ROWS, 64), lambda i: (i, 0))],
        out_shape=[jax.ShapeDtypeStruct((N, 64), jnp.float32),
                   jax.ShapeDtypeStruct((N, 128), jnp.float32),
                   jax.ShapeDtypeStruct((N, 64), jnp.float32)],
    )(x, w, b)


GPAIR = GIT // 2


def _gather_body(q_hbm, kv_hbm, dst_hbm, src_hbm, qd_hbm, kvs_hbm,
                 dv0, dv1, sv0, sv1, qr0, qr1, kr0, kr1,
                 sA0, sA1, sB0, sB1, sC0, sC1):
    cid = lax.axis_index("c")
    sid = lax.axis_index("s")
    wid = sid * _NC + cid
    base = wid * EPW
    DV = (dv0, dv1)
    SV = (sv0, sv1)
    QR = (qr0, qr1)
    KR = (kr0, kr1)
    SA = (sA0, sA1)
    SB = (sB0, sB1)
    SC = (sC0, sC1)

    def issue_a(t, s):
        off = base + t * GCH
        pltpu.async_copy(dst_hbm.at[pl.ds(off, GCH)], DV[s], SA[s])
        pltpu.async_copy(src_hbm.at[pl.ds(off, GCH)], SV[s], SA[s])

    def wait_a(s):
        pltpu.make_async_copy(dst_hbm.at[pl.ds(0, GCH)], DV[s], SA[s]).wait()
        pltpu.make_async_copy(src_hbm.at[pl.ds(0, GCH)], SV[s], SA[s]).wait()

    def issue_b(s):
        pltpu.async_copy(q_hbm.at[DV[s]], QR[s], SB[s])
        pltpu.async_copy(kv_hbm.at[SV[s]], KR[s], SB[s])

    def wait_b(s):
        pltpu.make_async_copy(q_hbm.at[DV[s]], QR[s], SB[s]).wait()
        pltpu.make_async_copy(kv_hbm.at[SV[s]], KR[s], SB[s]).wait()

    def issue_c(t, s):
        off = base + t * GCH
        pltpu.async_copy(QR[s], qd_hbm.at[pl.ds(off, GCH)], SC[s])
        pltpu.async_copy(KR[s], kvs_hbm.at[pl.ds(off, GCH)], SC[s])

    def wait_c(s):
        pltpu.make_async_copy(QR[s], qd_hbm.at[pl.ds(0, GCH)], SC[s]).wait()
        pltpu.make_async_copy(KR[s], kvs_hbm.at[pl.ds(0, GCH)], SC[s]).wait()

    issue_a(0, 0)
    issue_a(1, 1)

    def body(u, carry):
        t0 = u * 2
        wait_a(0)

        @pl.when(u > 0)
        def _():
            wait_c(0)

        issue_b(0)
        wait_a(1)

        @pl.when(u > 0)
        def _():
            wait_c(1)

        issue_b(1)
        wait_b(0)
        issue_c(t0, 0)
        issue_a(t0 + 2, 0)
        wait_b(1)
        issue_c(t0 + 1, 1)

        @pl.when(u < GPAIR - 1)
        def _():
            issue_a(t0 + 3, 1)

        return carry

    lax.fori_loop(0, GPAIR, body, 0)
    wait_a(0)
    wait_c(0)
    issue_b(0)
    wait_b(0)
    issue_c(GIT - 1, 0)
    wait_c(1)
    wait_c(0)


_gather = pl.kernel(
    _gather_body,
    out_type=[jax.ShapeDtypeStruct((E, 64), jnp.float32),
              jax.ShapeDtypeStruct((E, 128), jnp.float32)],
    mesh=_sc_mesh,
    scratch_types=[pltpu.VMEM((GCH,), jnp.int32),
                   pltpu.VMEM((GCH,), jnp.int32),
                   pltpu.VMEM((GCH,), jnp.int32),
                   pltpu.VMEM((GCH,), jnp.int32),
                   pltpu.VMEM((GCH, 64), jnp.float32),
                   pltpu.VMEM((GCH, 64), jnp.float32),
                   pltpu.VMEM((GCH, 128), jnp.float32),
                   pltpu.VMEM((GCH, 128), jnp.float32),
                   pltpu.SemaphoreType.DMA,
                   pltpu.SemaphoreType.DMA,
                   pltpu.SemaphoreType.DMA,
                   pltpu.SemaphoreType.DMA,
                   pltpu.SemaphoreType.DMA,
                   pltpu.SemaphoreType.DMA],
    compiler_params=_sc_params,
)


def _edge_body(qd_ref, kvs_ref, ms_ref, me_ref, ex_ref, ua_ref, ub_ref):
    qd = qd_ref[...]
    k = kvs_ref[:, :64]
    v = kvs_ref[:, 64:]
    alpha = jnp.dot(qd * k, ms_ref[...],
                    preferred_element_type=jnp.float32, precision=lax.Precision.HIGHEST) * (1.0 / np.sqrt(8.0))
    ex = jnp.exp(alpha)
    ex_ref[...] = ex
    u = v * jnp.dot(ex, me_ref[...], preferred_element_type=jnp.float32, precision=lax.Precision.HIGHEST)
    ua_ref[...] = u[:, :32]
    ub_ref[...] = u[:, 32:]


def _edge(qd, kvs, msum, mexp):
    return pl.pallas_call(
        _edge_body,
        grid=(E // BE,),
        in_specs=[pl.BlockSpec((BE, 64), lambda i: (i, 0)),
                  pl.BlockSpec((BE, 128), lambda i: (i, 0)),
                  pl.BlockSpec((64, 8), lambda i: (0, 0)),
                  pl.BlockSpec((8, 64), lambda i: (0, 0))],
        out_specs=[pl.BlockSpec((BE, 8), lambda i: (i, 0)),
                   pl.BlockSpec((BE, 32), lambda i: (i, 0)),
                   pl.BlockSpec((BE, 32), lambda i: (i, 0))],
        out_shape=[jax.ShapeDtypeStruct((E, 8), jnp.float32),
                   jax.ShapeDtypeStruct((E, 32), jnp.float32),
                   jax.ShapeDtypeStruct((E, 32), jnp.float32)],
    )(qd, kvs, msum, mexp)


SCH = 200
SIT = EPT // SCH
DIT = EPW // SCH
DPAIR = DIT // 2


def _zero_acc(z_hbm, zbuf, acc, sid):
    pltpu.sync_copy(z_hbm, zbuf)

    def zbody(t, carry):
        j = sid + t * _NS

        @pl.when(j < ZNB)
        def _():
            pltpu.sync_copy(zbuf, acc.at[pl.ds(j * ZCH, ZCH)])
        return carry

    lax.fori_loop(0, ZPT, zbody, 0)


def _write_acc(acc, wbuf, out0_hbm, out1_hbm, cid, sid):
    def wbody(t, carry):
        j = sid + t * _NS

        @pl.when(j < ZNB)
        def _():
            r0 = j * ZCH
            pltpu.sync_copy(acc.at[pl.ds(r0, ZCH)], wbuf)

            @pl.when(cid == 0)
            def _():
                pltpu.sync_copy(wbuf, out0_hbm.at[pl.ds(r0, ZCH)])

            @pl.when(cid == 1)
            def _():
                pltpu.sync_copy(wbuf, out1_hbm.at[pl.ds(r0, ZCH)])
        return carry

    lax.fori_loop(0, ZPT, wbody, 0)


def _aggk_body(ua_hbm, ub_hbm, dst_hbm, z32_hbm, a0_hbm, a1_hbm,
               iv0, iv1, ur0, ur1, zb32, acc32, sA0, sA1, sB0, sB1):
    cid = lax.axis_index("c")
    sid = lax.axis_index("s")
    IV = (iv0, iv1)
    UR = (ur0, ur1)
    SA = (sA0, sA1)
    SB = (sB0, sB1)
    _zero_acc(z32_hbm, zb32, acc32, sid)
    plsc.subcore_barrier()

    abase = sid * EPT

    def a_issue(t, s):
        off = abase + t * SCH
        pltpu.async_copy(dst_hbm.at[pl.ds(off, SCH)], IV[s], SA[s])

        @pl.when(cid == 0)
        def _():
            pltpu.async_copy(ua_hbm.at[pl.ds(off, SCH)], UR[s], SA[s])

        @pl.when(cid == 1)
        def _():
            pltpu.async_copy(ub_hbm.at[pl.ds(off, SCH)], UR[s], SA[s])

    def a_wait(s):
        pltpu.make_async_copy(dst_hbm.at[pl.ds(0, SCH)], IV[s], SA[s]).wait()
        pltpu.make_async_copy(ua_hbm.at[pl.ds(0, SCH)], UR[s], SA[s]).wait()

    def b_issue(s):
        pltpu.async_copy(UR[s], acc32.at[IV[s]], SB[s], add=True)

    def b_wait(s):
        pltpu.make_async_copy(UR[s], acc32.at[IV[s]], SB[s]).wait()

    a_issue(0, 0)
    a_issue(1, 1)

    def abody(u, carry):
        t0 = u * 2
        a_wait(0)
        b_issue(0)
        a_wait(1)
        b_issue(1)
        b_wait(0)

        @pl.when(u < SIT // 2 - 1)
        def _():
            a_issue(t0 + 2, 0)

        b_wait(1)

        @pl.when(u < SIT // 2 - 1)
        def _():
            a_issue(t0 + 3, 1)

        return carry

    lax.fori_loop(0, SIT // 2, abody, 0)
    plsc.subcore_barrier()
    _write_acc(acc32, zb32, a0_hbm, a1_hbm, cid, sid)


_aggk = pl.kernel(
    _aggk_body,
    out_type=[jax.ShapeDtypeStruct((N, 32), jnp.float32),
              jax.ShapeDtypeStruct((N, 32), jnp.float32)],
    mesh=_sc_mesh,
    scratch_types=[pltpu.VMEM((SCH,), jnp.int32),
                   pltpu.VMEM((SCH,), jnp.int32),
                   pltpu.VMEM((SCH, 32), jnp.float32),
                   pltpu.VMEM((SCH, 32), jnp.float32),
                   pltpu.VMEM((ZCH, 32), jnp.float32),
                   pltpu.VMEM_SHARED((N, 32), jnp.float32),
                   pltpu.SemaphoreType.DMA,
                   pltpu.SemaphoreType.DMA,
                   pltpu.SemaphoreType.DMA,
                   pltpu.SemaphoreType.DMA],
    compiler_params=_sc_params,
)


def _denomk_body(ex_hbm, dst_hbm, z8_hbm, d0_hbm, d1_hbm,
                 iv0, iv1, er0, er1, zb8, acc8, sA0, sA1, sB0, sB1):
    cid = lax.axis_index("c")
    sid = lax.axis_index("s")
    wid = sid * _NC + cid
    IV = (iv0, iv1)
    ER = (er0, er1)
    SA = (sA0, sA1)
    SB = (sB0, sB1)
    _zero_acc(z8_hbm, zb8, acc8, sid)
    plsc.subcore_barrier()

    dbase = wid * EPW

    def d_issue(t, s):
        off = dbase + t * SCH
        pltpu.async_copy(dst_hbm.at[pl.ds(off, SCH)], IV[s], SA[s])
        pltpu.async_copy(ex_hbm.at[pl.ds(off, SCH)], ER[s], SA[s])

    def d_wait(s):
        pltpu.make_async_copy(dst_hbm.at[pl.ds(0, SCH)], IV[s], SA[s]).wait()
        pltpu.make_async_copy(ex_hbm.at[pl.ds(0, SCH)], ER[s], SA[s]).wait()

    def e_issue(s):
        pltpu.async_copy(ER[s], acc8.at[IV[s]], SB[s], add=True)

    def e_wait(s):
        pltpu.make_async_copy(ER[s], acc8.at[IV[s]], SB[s]).wait()

    d_issue(0, 0)
    d_issue(1, 1)

    def dbody(u, carry):
        t0 = u * 2
        d_wait(0)
        e_issue(0)
        d_wait(1)
        e_issue(1)
        e_wait(0)
        d_issue(t0 + 2, 0)
        e_wait(1)

        @pl.when(u < DPAIR - 1)
        def _():
            d_issue(t0 + 3, 1)

        return carry

    lax.fori_loop(0, DPAIR, dbody, 0)
    d_wait(0)
    e_issue(0)
    e_wait(0)
    plsc.subcore_barrier()
    _write_acc(acc8, zb8, d0_hbm, d1_hbm, cid, sid)


_denomk = pl.kernel(
    _denomk_body,
    out_type=[jax.ShapeDtypeStruct((N, 8), jnp.float32),
              jax.ShapeDtypeStruct((N, 8), jnp.float32)],
    mesh=_sc_mesh,
    scratch_types=[pltpu.VMEM((SCH,), jnp.int32),
                   pltpu.VMEM((SCH,), jnp.int32),
                   pltpu.VMEM((SCH, 8), jnp.float32),
                   pltpu.VMEM((SCH, 8), jnp.float32),
                   pltpu.VMEM((ZCH, 8), jnp.float32),
                   pltpu.VMEM_SHARED((N, 8), jnp.float32),
                   pltpu.SemaphoreType.DMA,
                   pltpu.SemaphoreType.DMA,
                   pltpu.SemaphoreType.DMA,
                   pltpu.SemaphoreType.DMA],
    compiler_params=_sc_params,
)


def _post_a_body(a0_ref, a1_ref, d0_ref, d1_ref, s_ref, me_ref,
                 r_ref, ps_ref, pq_ref):
    d8 = jnp.maximum(d0_ref[...] + d1_ref[...], 1e-30)
    dexp = jnp.dot(d8, me_ref[...], preferred_element_type=jnp.float32, precision=lax.Precision.HIGHEST)
    agg = jnp.concatenate([a0_ref[...], a1_ref[...]], axis=1) / dexp
    r = jnp.maximum(agg + s_ref[...], 0.0)
    r_ref[...] = r
    ps_ref[...] = jnp.sum(r, axis=0, keepdims=True).reshape(1, 1, 64)
    pq_ref[...] = jnp.sum(r * r, axis=0, keepdims=True).reshape(1, 1, 64)


def _post_a(a0, a1, d0, d1, s, mexp):
    return pl.pallas_call(
        _post_a_body,
        grid=(NB,),
        in_specs=[pl.BlockSpec((BN_ROWS, 32), lambda i: (i, 0)),
                  pl.BlockSpec((BN_ROWS, 32), lambda i: (i, 0)),
                  pl.BlockSpec((BN_ROWS, 8), lambda i: (i, 0)),
                  pl.BlockSpec((BN_ROWS, 8), lambda i: (i, 0)),
                  pl.BlockSpec((BN_ROWS, 64), lambda i: (i, 0)),
                  pl.BlockSpec((8, 64), lambda i: (0, 0))],
        out_specs=[pl.BlockSpec((BN_ROWS, 64), lambda i: (i, 0)),
                   pl.BlockSpec((1, 1, 64), lambda i: (i, 0, 0)),
                   pl.BlockSpec((1, 1, 64), lambda i: (i, 0, 0))],
        out_shape=[jax.ShapeDtypeStruct((N, 64), jnp.float32),
                   jax.ShapeDtypeStruct((NB, 1, 64), jnp.float32),
                   jax.ShapeDtypeStruct((NB, 1, 64), jnp.float32)],
    )(a0, a1, d0, d1, s, mexp)


def _post_b_body_res(r_ref, ps_ref, pq_ref, g_ref, b_ref, xp_ref, o_ref):
    mu = jnp.sum(ps_ref[...].reshape(NB, 64), axis=0, keepdims=True) * (1.0 / N)
    msq = jnp.sum(pq_ref[...].reshape(NB, 64), axis=0, keepdims=True) * (1.0 / N)
    var = msq - mu * mu
    inv = lax.rsqrt(var + 1e-5)
    o_ref[...] = ((r_ref[...] - mu) * inv * g_ref[...] + b_ref[...]
                  + xp_ref[...])


def _post_b_body_nores(r_ref, ps_ref, pq_ref, g_ref, b_ref, o_ref):
    mu = jnp.sum(ps_ref[...].reshape(NB, 64), axis=0, keepdims=True) * (1.0 / N)
    msq = jnp.sum(pq_ref[...].reshape(NB, 64), axis=0, keepdims=True) * (1.0 / N)
    var = msq - mu * mu
    inv = lax.rsqrt(var + 1e-5)
    o_ref[...] = (r_ref[...] - mu) * inv * g_ref[...] + b_ref[...]


def _post_b(r, ps, pq, gamma, beta, xprev):
    body = _post_b_body_nores if xprev is None else _post_b_body_res
    in_specs = [pl.BlockSpec((BN_ROWS, 64), lambda i: (i, 0)),
                pl.BlockSpec((NB, 1, 64), lambda i: (0, 0, 0)),
                pl.BlockSpec((NB, 1, 64), lambda i: (0, 0, 0)),
                pl.BlockSpec((1, 64), lambda i: (0, 0)),
                pl.BlockSpec((1, 64), lambda i: (0, 0))]
    args = [r, ps, pq, gamma, beta]
    if xprev is not None:
        in_specs.append(pl.BlockSpec((BN_ROWS, 64), lambda i: (i, 0)))
        args.append(xprev)
    return pl.pallas_call(
        body,
        grid=(NB,),
        in_specs=in_specs,
        out_specs=pl.BlockSpec((BN_ROWS, 64), lambda i: (i, 0)),
        out_shape=jax.ShapeDtypeStruct((N, 64), jnp.float32),
    )(*args)


def _pool_body(x_ref, b_ref, s_ref, c_ref):
    i = pl.program_id(0)

    @pl.when(i == 0)
    def _():
        s_ref[...] = jnp.zeros_like(s_ref)
        c_ref[...] = jnp.zeros_like(c_ref)

    bvals = b_ref[...]
    cols = lax.broadcasted_iota(jnp.int32, (BN_ROWS, B), 1)
    onehot = jnp.where(bvals == cols, 1.0, 0.0)
    s_ref[...] += lax.dot_general(onehot, x_ref[...],
                                  (((0,), (0,)), ((), ())),
                                  preferred_element_type=jnp.float32,
                                  precision=lax.Precision.HIGHEST)
    c_ref[...] += lax.dot_general(onehot, jnp.ones((BN_ROWS, 1), jnp.float32),
                                  (((0,), (0,)), ((), ())),
                                  preferred_element_type=jnp.float32,
                                  precision=lax.Precision.HIGHEST)


def _pool(x, batch2d):
    return pl.pallas_call(
        _pool_body,
        grid=(NB,),
        in_specs=[pl.BlockSpec((BN_ROWS, 64), lambda i: (i, 0)),
                  pl.BlockSpec((BN_ROWS, 1), lambda i: (i, 0))],
        out_specs=[pl.BlockSpec((B, 64), lambda i: (0, 0)),
                   pl.BlockSpec((B, 1), lambda i: (0, 0))],
        out_shape=[jax.ShapeDtypeStruct((B, 64), jnp.float32),
                   jax.ShapeDtypeStruct((B, 1), jnp.float32)],
    )(x, batch2d)


def _final_body(s_ref, c_ref, w_ref, b_ref, o_ref, xm_ref):
    xm = s_ref[...] / jnp.maximum(c_ref[...], 1.0)
    xm_ref[...] = xm
    o_ref[...] = jnp.dot(xm, w_ref[...],
                         preferred_element_type=jnp.float32) + b_ref[...]


def _final(ssum, cnt, w, b):
    return pl.pallas_call(
        _final_body,
        grid=(1,),
        in_specs=[pl.BlockSpec((B, 64), lambda i: (0, 0)),
                  pl.BlockSpec((B, 1), lambda i: (0, 0)),
                  pl.BlockSpec((64, 1), lambda i: (0, 0)),
                  pl.BlockSpec((1, 1), lambda i: (0, 0))],
        out_specs=[pl.BlockSpec((B, 1), lambda i: (0, 0)),
                   pl.BlockSpec((B, 64), lambda i: (0, 0))],
        out_shape=[jax.ShapeDtypeStruct((B, 1), jnp.float32),
                   jax.ShapeDtypeStruct((B, 64), jnp.float32)],
    )(ssum, cnt, w, b)


def kernel(X, edge_index, batch, params):
    src = edge_index[0]
    dst = edge_index[1]
    msum = jnp.asarray(_MSUM)
    mexp = jnp.asarray(_MEXP)
    z8 = jnp.zeros((ZCH, 8), jnp.float32)
    z32 = jnp.zeros((ZCH, 32), jnp.float32)

    x = X
    for l in range(5):
        p = params['convs'][l]
        w_all = jnp.concatenate([p['Wq'], p['Wk'], p['Wv'], p['Ws']], axis=1)
        b_all = jnp.concatenate([p['bq'], p['bk'], p['bv'], p['bs']]).reshape(1, 256)
        q, kv, s = _qkvs(x, w_all, b_all)
        qd, kvs = _gather(q, kv, dst, src)
        ex8, ua, ub = _edge(qd, kvs, msum, mexp)
        a0, a1 = _aggk(ua, ub, dst, z32)
        d0, d1 = _denomk(ex8, dst, z8)
        r, ps, pq = _post_a(a0, a1, d0, d1, s, mexp)
        bn = params['bns'][l]
        x = _post_b(r, ps, pq, bn['gamma'].reshape(1, 64), bn['beta'].reshape(1, 64),
                    xprev=(x if l > 0 else None))

    ssum, cnt = _pool(x, batch.reshape(N, 1))
    out, xm = _final(ssum, cnt, params['reg_W'], params['reg_b'].reshape(1, 1))
    return (out, xm)
